# baseline (speedup 1.0000x reference)
import functools

import jax
import jax.numpy as jnp
from jax import lax
from jax.experimental import pallas as pl
from jax.experimental.pallas import tpu as pltpu
from jax.experimental.pallas import tpu_sc as plsc

EMB_D = 128
N_NODE = 10000
N_EDGE = 160000
N_ROUND = 8

_TILES = 16
_CHUNK = 128
_EPT = N_EDGE // _TILES
_NCH = -(-_EPT // _CHUNK)
_PAD = _NCH * _CHUNK - _EPT
_DUMMY = N_NODE
_RCH = 80
_NRC = N_NODE // _RCH

_sc_mesh = plsc.VectorSubcoreMesh(core_axis_name="c", subcore_axis_name="s")


def _sc_edge_body(tbl, gp, sp, gn, sn, out, acc, gbuf, sbuf,
                  rows0, rows1, gs0, gs1, ss0, ss1):
    c = lax.axis_index("c")
    s = lax.axis_index("s")
    nrc_mine = jnp.where(s < _NRC % _TILES, _NRC // _TILES + 1,
                         _NRC // _TILES)

    zero = jnp.zeros((32,), jnp.bfloat16)

    def _zrow(r, carry):
        for j in range(EMB_D // 32):
            rows0[r, pl.ds(j * 32, 32)] = zero
        return carry

    lax.fori_loop(0, _CHUNK, _zrow, 0)

    def _zero_chunk(t, carry):
        k = s + _TILES * t
        pltpu.sync_copy(rows0.at[pl.ds(0, _RCH)],
                        acc.at[pl.ds(k * _RCH, _RCH)])
        return carry

    lax.fori_loop(0, nrc_mine, _zero_chunk, 0)
    plsc.subcore_barrier()

    def _g_start(k, buf, sem):
        pltpu.async_copy(tbl.at[gbuf.at[k]], buf, sem)

    def _wait(buf, sem):
        pltpu.make_async_copy(tbl.at[pl.ds(0, _CHUNK)], buf, sem).wait()

    def _edges(gidx, sidx):
        pltpu.sync_copy(gidx.at[s], gbuf)
        pltpu.sync_copy(sidx.at[s], sbuf)
        _g_start(0, rows0, gs0)

        def body(i, carry):
            c0 = 2 * i
            _g_start(c0 + 1, rows1, gs1)
            _wait(rows0, gs0)
            pltpu.sync_copy(rows0, acc.at[sbuf.at[c0]], add=True)
            _g_start(c0 + 2, rows0, gs0)
            _wait(rows1, gs1)
            pltpu.sync_copy(rows1, acc.at[sbuf.at[c0 + 1]], add=True)
            return carry

        lax.fori_loop(0, (_NCH - 1) // 2, body, 0)
        _wait(rows0, gs0)
        pltpu.sync_copy(rows0, acc.at[sbuf.at[_NCH - 1]], add=True)

    @pl.when(c == 0)
    def _():
        _edges(gp, sp)

    @pl.when(c == 1)
    def _():
        _edges(gn, sn)

    plsc.subcore_barrier()

    def _out_chunk(t, carry):
        k = s + _TILES * t
        pltpu.sync_copy(acc.at[pl.ds(k * _RCH, _RCH)],
                        out.at[pl.ds(c * N_NODE + k * _RCH, _RCH)])
        return carry

    lax.fori_loop(0, nrc_mine, _out_chunk, 0)


_sc_edge_pass = functools.partial(
    pl.kernel,
    out_type=jax.ShapeDtypeStruct((2 * N_NODE, EMB_D), jnp.bfloat16),
    mesh=_sc_mesh,
    scratch_types=[
        pltpu.VMEM_SHARED((N_NODE + 16, EMB_D), jnp.bfloat16),
        pltpu.VMEM((_NCH, _CHUNK), jnp.int32),
        pltpu.VMEM((_NCH, _CHUNK), jnp.int32),
        pltpu.VMEM((_CHUNK, EMB_D), jnp.bfloat16),
        pltpu.VMEM((_CHUNK, EMB_D), jnp.bfloat16),
        pltpu.SemaphoreType.DMA,
        pltpu.SemaphoreType.DMA,
        pltpu.SemaphoreType.DMA,
        pltpu.SemaphoreType.DMA,
    ],
    compiler_params=pltpu.CompilerParams(use_tc_tiling_on_sc=False),
)(_sc_edge_body)


_BR = 2000
_NB = N_NODE // _BR


def _mlp3(x, w1, b1, w2, b2, w3, b3):
    h = jnp.maximum(jnp.dot(x, w1, preferred_element_type=jnp.float32) + b1, 0.0)
    h = jnp.maximum(jnp.dot(h, w2, preferred_element_type=jnp.float32) + b2, 0.0)
    return jnp.dot(h, w3, preferred_element_type=jnp.float32) + b3


def _msg_body(xl, xc, wl1, bl1, wl2, bl2, wl3, bl3, wc1, bc1, wc2, bc2,
              wc3, bc3, ol, oc):
    ml = _mlp3(xl[0], wl1[0], bl1[0], wl2[0], bl2[0], wl3[0], bl3[0])
    mc = _mlp3(xc[0], wc1[0], bc1[0], wc2[0], bc2[0], wc3[0], bc3[0])
    ol[...] = ml.astype(jnp.bfloat16)
    oc[...] = mc.astype(jnp.bfloat16)


def _msg_call(S, wl, bl, wc, bc):
    w_spec = pl.BlockSpec((1, EMB_D, EMB_D), lambda i, b: (i, 0, 0))
    b_spec = pl.BlockSpec((1, 1, EMB_D), lambda i, b: (i, 0, 0))
    return pl.pallas_call(
        _msg_body,
        grid=(2, _NB),
        in_specs=[
            pl.BlockSpec((1, _BR, EMB_D), lambda i, b: (i, b, 0)),
            pl.BlockSpec((1, _BR, EMB_D), lambda i, b: (2, b, 0)),
            w_spec, b_spec, w_spec, b_spec, w_spec, b_spec,
            w_spec, b_spec, w_spec, b_spec, w_spec, b_spec,
        ],
        out_specs=[
            pl.BlockSpec((_BR, EMB_D), lambda i, b: (i * _NB + b, 0)),
            pl.BlockSpec((_BR, EMB_D), lambda i, b: (i * _NB + b, 0)),
        ],
        out_shape=[
            jax.ShapeDtypeStruct((2 * N_NODE, EMB_D), jnp.bfloat16),
            jax.ShapeDtypeStruct((2 * N_NODE, EMB_D), jnp.bfloat16),
        ],
    )(S, S, wl[0], bl[0], wl[1], bl[1], wl[2], bl[2],
      wc[0], bc[0], wc[1], bc[1], wc[2], bc[2])


def _lstm_body(c2l_a, c2l_b, l2c_a, l2c_b, s_ref, c_ref, wih, whh, bb,
               so_ref, co_ref):
    i = pl.program_id(0)
    xl = c2l_a[...].astype(jnp.float32) + c2l_b[...].astype(jnp.float32)
    xc = l2c_a[...].astype(jnp.float32) + l2c_b[...].astype(jnp.float32)
    x = jnp.where(i < 2, xl, xc)
    h = s_ref[0]
    cc = c_ref[0]
    gates = (jnp.dot(x, wih[0], preferred_element_type=jnp.float32)
             + jnp.dot(h, whh[0], preferred_element_type=jnp.float32)
             + bb[0])
    ig = gates[:, 0 * EMB_D:1 * EMB_D]
    fg = gates[:, 1 * EMB_D:2 * EMB_D]
    gg = gates[:, 2 * EMB_D:3 * EMB_D]
    og = gates[:, 3 * EMB_D:4 * EMB_D]
    c_new = jax.nn.sigmoid(fg) * cc + jax.nn.sigmoid(ig) * jnp.tanh(gg)
    co_ref[0] = c_new
    so_ref[0] = jax.nn.sigmoid(og) * jnp.tanh(c_new)


def _lstm_call(l2c, c2l, S, C, wih, whh, bb):
    half_spec_a = pl.BlockSpec((_BR, EMB_D), lambda i, b: (b, 0))
    half_spec_b = pl.BlockSpec((_BR, EMB_D), lambda i, b: (_NB + b, 0))
    st_spec = pl.BlockSpec((1, _BR, EMB_D), lambda i, b: (i, b, 0))
    return pl.pallas_call(
        _lstm_body,
        grid=(3, _NB),
        in_specs=[
            half_spec_a, half_spec_b, half_spec_a, half_spec_b,
            st_spec, st_spec,
            pl.BlockSpec((1, EMB_D, 4 * EMB_D), lambda i, b: (i // 2, 0, 0)),
            pl.BlockSpec((1, EMB_D, 4 * EMB_D), lambda i, b: (i // 2, 0, 0)),
            pl.BlockSpec((1, 1, 4 * EMB_D), lambda i, b: (i // 2, 0, 0)),
        ],
        out_specs=[st_spec, st_spec],
        out_shape=[
            jax.ShapeDtypeStruct((3, N_NODE, EMB_D), jnp.float32),
            jax.ShapeDtypeStruct((3, N_NODE, EMB_D), jnp.float32),
        ],
    )(c2l, c2l, l2c, l2c, S, C, wih, whh, bb)


def kernel(pos_l_emb, neg_l_emb, c_emb, pos_edge_index, neg_edge_index, params):
    def _tile_idx(e, fill):
        e2 = e.reshape(_TILES, _EPT)
        pad = jnp.full((_TILES, _PAD), fill, jnp.int32)
        return jnp.concatenate([e2, pad], axis=1).reshape(_TILES, _NCH, _CHUNK)

    ps_g = _tile_idx(pos_edge_index[0], 0)
    ps_s = _tile_idx(pos_edge_index[0], _DUMMY)
    pd_g = _tile_idx(pos_edge_index[1], 0)
    pd_s = _tile_idx(pos_edge_index[1], _DUMMY)
    ns_bg = _tile_idx(neg_edge_index[0] + N_NODE, 0)
    ns_s = _tile_idx(neg_edge_index[0], _DUMMY)
    nd_bg = _tile_idx(neg_edge_index[1] + N_NODE, 0)
    nd_s = _tile_idx(neg_edge_index[1], _DUMMY)

    wl = [jnp.stack([params['pos_l_mlp']['W%d' % k].T,
                     params['neg_l_mlp']['W%d' % k].T]) for k in (1, 2, 3)]
    bl = [jnp.stack([params['pos_l_mlp']['b%d' % k],
                     params['neg_l_mlp']['b%d' % k]])[:, None, :] for k in (1, 2, 3)]
    wc = [jnp.stack([params['pos_c_mlp']['W%d' % k].T,
                     params['neg_c_mlp']['W%d' % k].T]) for k in (1, 2, 3)]
    bc = [jnp.stack([params['pos_c_mlp']['b%d' % k],
                     params['neg_c_mlp']['b%d' % k]])[:, None, :] for k in (1, 2, 3)]
    wih = jnp.stack([params['l_lstm']['W_ih'].T, params['c_lstm']['W_ih'].T])
    whh = jnp.stack([params['l_lstm']['W_hh'].T, params['c_lstm']['W_hh'].T])
    bb = jnp.stack([params['l_lstm']['b_ih'] + params['l_lstm']['b_hh'],
                    params['c_lstm']['b_ih'] + params['c_lstm']['b_hh']])[:, None, :]

    S = jnp.stack([pos_l_emb, neg_l_emb, c_emb])
    C = jnp.zeros_like(S)
    for _ in range(N_ROUND):
        msg_l, msg_c = _msg_call(S, wl, bl, wc, bc)
        l2c = _sc_edge_pass(msg_l, ps_g, pd_s, ns_bg, nd_s)
        c2l = _sc_edge_pass(msg_c, pd_g, ps_s, nd_bg, ns_s)
        S, C = _lstm_call(l2c, c2l, S, C, wih, whh, bb)
    return S.reshape(3 * N_NODE, EMB_D)


# speedup vs baseline: 1.4884x; 1.4884x over previous
import functools

import jax
import jax.numpy as jnp
from jax import lax
from jax.experimental import pallas as pl
from jax.experimental.pallas import tpu as pltpu
from jax.experimental.pallas import tpu_sc as plsc

EMB_D = 128
N_NODE = 10000
N_EDGE = 160000
N_ROUND = 8

_TILES = 16
_CHUNK = 112
_EPT = N_EDGE // _TILES
_NCH = -(-_EPT // _CHUNK)
_PAD = _NCH * _CHUNK - _EPT
_DUMMY = N_NODE
_RCH = 80
_NRC = N_NODE // _RCH

_sc_mesh = plsc.VectorSubcoreMesh(core_axis_name="c", subcore_axis_name="s")


def _sc_edge_body(tbl, gp, sp, gn, sn, out, acc, gbuf, sbuf,
                  rows0, rows1, gs0, gs1, ss0, ss1):
    c = lax.axis_index("c")
    s = lax.axis_index("s")
    nrc_mine = jnp.where(s < _NRC % _TILES, _NRC // _TILES + 1,
                         _NRC // _TILES)

    zero = jnp.zeros((32,), jnp.bfloat16)

    def _zrow(r, carry):
        for j in range(EMB_D // 32):
            rows0[r, pl.ds(j * 32, 32)] = zero
        return carry

    lax.fori_loop(0, _CHUNK, _zrow, 0)

    def _zero_chunk(t, carry):
        k = s + _TILES * t
        pltpu.sync_copy(rows0.at[pl.ds(0, _RCH)],
                        acc.at[pl.ds(k * _RCH, _RCH)])
        return carry

    lax.fori_loop(0, nrc_mine, _zero_chunk, 0)
    plsc.subcore_barrier()

    def _g_start(k, buf, sem):
        pltpu.async_copy(tbl.at[gbuf.at[k]], buf, sem)

    def _wait(buf, sem):
        pltpu.make_async_copy(tbl.at[pl.ds(0, _CHUNK)], buf, sem).wait()

    def _edges(gidx, sidx):
        pltpu.sync_copy(gidx.at[s], gbuf)
        pltpu.sync_copy(sidx.at[s], sbuf)
        _g_start(0, rows0, gs0)

        def body(i, carry):
            c0 = 2 * i
            _g_start(c0 + 1, rows1, gs1)
            _wait(rows0, gs0)
            pltpu.sync_copy(rows0, acc.at[sbuf.at[c0]], add=True)
            _g_start(c0 + 2, rows0, gs0)
            _wait(rows1, gs1)
            pltpu.sync_copy(rows1, acc.at[sbuf.at[c0 + 1]], add=True)
            return carry

        lax.fori_loop(0, (_NCH - 1) // 2, body, 0)
        _wait(rows0, gs0)
        pltpu.sync_copy(rows0, acc.at[sbuf.at[_NCH - 1]], add=True)

    @pl.when(c == 0)
    def _():
        _edges(gp, sp)

    @pl.when(c == 1)
    def _():
        _edges(gn, sn)

    plsc.subcore_barrier()

    def _out_chunk(t, carry):
        k = s + _TILES * t
        pltpu.sync_copy(acc.at[pl.ds(k * _RCH, _RCH)],
                        out.at[pl.ds(c * N_NODE + k * _RCH, _RCH)])
        return carry

    lax.fori_loop(0, nrc_mine, _out_chunk, 0)


_sc_edge_pass = functools.partial(
    pl.kernel,
    out_type=jax.ShapeDtypeStruct((2 * N_NODE, EMB_D), jnp.bfloat16),
    mesh=_sc_mesh,
    scratch_types=[
        pltpu.VMEM_SHARED((N_NODE + 16, EMB_D), jnp.bfloat16),
        pltpu.VMEM((_NCH, _CHUNK), jnp.int32),
        pltpu.VMEM((_NCH, _CHUNK), jnp.int32),
        pltpu.VMEM((_CHUNK, EMB_D), jnp.bfloat16),
        pltpu.VMEM((_CHUNK, EMB_D), jnp.bfloat16),
        pltpu.SemaphoreType.DMA,
        pltpu.SemaphoreType.DMA,
        pltpu.SemaphoreType.DMA,
        pltpu.SemaphoreType.DMA,
    ],
    compiler_params=pltpu.CompilerParams(use_tc_tiling_on_sc=False),
)(_sc_edge_body)


_BR = 2000
_NB = N_NODE // _BR


def _mlp3(x, w1, b1, w2, b2, w3, b3):
    h = jnp.maximum(jnp.dot(x, w1, preferred_element_type=jnp.float32) + b1, 0.0)
    h = jnp.maximum(jnp.dot(h, w2, preferred_element_type=jnp.float32) + b2, 0.0)
    return jnp.dot(h, w3, preferred_element_type=jnp.float32) + b3


def _msg_body(xl, xc, wl1, bl1, wl2, bl2, wl3, bl3, wc1, bc1, wc2, bc2,
              wc3, bc3, ol, oc):
    ml = _mlp3(xl[0], wl1[0], bl1[0], wl2[0], bl2[0], wl3[0], bl3[0])
    mc = _mlp3(xc[0], wc1[0], bc1[0], wc2[0], bc2[0], wc3[0], bc3[0])
    ol[...] = ml.astype(jnp.bfloat16)
    oc[...] = mc.astype(jnp.bfloat16)


def _msg_call(S, wl, bl, wc, bc):
    w_spec = pl.BlockSpec((1, EMB_D, EMB_D), lambda i, b: (i, 0, 0))
    b_spec = pl.BlockSpec((1, 1, EMB_D), lambda i, b: (i, 0, 0))
    return pl.pallas_call(
        _msg_body,
        grid=(2, _NB),
        in_specs=[
            pl.BlockSpec((1, _BR, EMB_D), lambda i, b: (i, b, 0)),
            pl.BlockSpec((1, _BR, EMB_D), lambda i, b: (2, b, 0)),
            w_spec, b_spec, w_spec, b_spec, w_spec, b_spec,
            w_spec, b_spec, w_spec, b_spec, w_spec, b_spec,
        ],
        out_specs=[
            pl.BlockSpec((_BR, EMB_D), lambda i, b: (i * _NB + b, 0)),
            pl.BlockSpec((_BR, EMB_D), lambda i, b: (i * _NB + b, 0)),
        ],
        out_shape=[
            jax.ShapeDtypeStruct((2 * N_NODE, EMB_D), jnp.bfloat16),
            jax.ShapeDtypeStruct((2 * N_NODE, EMB_D), jnp.bfloat16),
        ],
    )(S, S, wl[0], bl[0], wl[1], bl[1], wl[2], bl[2],
      wc[0], bc[0], wc[1], bc[1], wc[2], bc[2])


def _lstm_body(c2l_a, c2l_b, l2c_a, l2c_b, s_ref, c_ref, wih, whh, bb,
               so_ref, co_ref):
    i = pl.program_id(0)
    xl = c2l_a[...].astype(jnp.float32) + c2l_b[...].astype(jnp.float32)
    xc = l2c_a[...].astype(jnp.float32) + l2c_b[...].astype(jnp.float32)
    x = jnp.where(i < 2, xl, xc)
    h = s_ref[0]
    cc = c_ref[0]
    gates = (jnp.dot(x, wih[0], preferred_element_type=jnp.float32)
             + jnp.dot(h, whh[0], preferred_element_type=jnp.float32)
             + bb[0])
    ig = gates[:, 0 * EMB_D:1 * EMB_D]
    fg = gates[:, 1 * EMB_D:2 * EMB_D]
    gg = gates[:, 2 * EMB_D:3 * EMB_D]
    og = gates[:, 3 * EMB_D:4 * EMB_D]
    c_new = jax.nn.sigmoid(fg) * cc + jax.nn.sigmoid(ig) * jnp.tanh(gg)
    co_ref[0] = c_new
    so_ref[0] = jax.nn.sigmoid(og) * jnp.tanh(c_new)


def _lstm_call(l2c, c2l, S, C, wih, whh, bb):
    half_spec_a = pl.BlockSpec((_BR, EMB_D), lambda i, b: (b, 0))
    half_spec_b = pl.BlockSpec((_BR, EMB_D), lambda i, b: (_NB + b, 0))
    st_spec = pl.BlockSpec((1, _BR, EMB_D), lambda i, b: (i, b, 0))
    return pl.pallas_call(
        _lstm_body,
        grid=(3, _NB),
        in_specs=[
            half_spec_a, half_spec_b, half_spec_a, half_spec_b,
            st_spec, st_spec,
            pl.BlockSpec((1, EMB_D, 4 * EMB_D), lambda i, b: (i // 2, 0, 0)),
            pl.BlockSpec((1, EMB_D, 4 * EMB_D), lambda i, b: (i // 2, 0, 0)),
            pl.BlockSpec((1, 1, 4 * EMB_D), lambda i, b: (i // 2, 0, 0)),
        ],
        out_specs=[st_spec, st_spec],
        out_shape=[
            jax.ShapeDtypeStruct((3, N_NODE, EMB_D), jnp.float32),
            jax.ShapeDtypeStruct((3, N_NODE, EMB_D), jnp.float32),
        ],
    )(c2l, c2l, l2c, l2c, S, C, wih, whh, bb)


def kernel(pos_l_emb, neg_l_emb, c_emb, pos_edge_index, neg_edge_index, params):
    def _tile_idx(e, fill):
        e2 = e.reshape(_TILES, _EPT)
        pad = jnp.full((_TILES, _PAD), fill, jnp.int32)
        return jnp.concatenate([e2, pad], axis=1).reshape(_TILES, _NCH, _CHUNK)

    ps_g = _tile_idx(pos_edge_index[0], 0)
    ps_s = _tile_idx(pos_edge_index[0], _DUMMY)
    pd_g = _tile_idx(pos_edge_index[1], 0)
    pd_s = _tile_idx(pos_edge_index[1], _DUMMY)
    ns_bg = _tile_idx(neg_edge_index[0] + N_NODE, 0)
    ns_s = _tile_idx(neg_edge_index[0], _DUMMY)
    nd_bg = _tile_idx(neg_edge_index[1] + N_NODE, 0)
    nd_s = _tile_idx(neg_edge_index[1], _DUMMY)

    wl = [jnp.stack([params['pos_l_mlp']['W%d' % k].T,
                     params['neg_l_mlp']['W%d' % k].T]) for k in (1, 2, 3)]
    bl = [jnp.stack([params['pos_l_mlp']['b%d' % k],
                     params['neg_l_mlp']['b%d' % k]])[:, None, :] for k in (1, 2, 3)]
    wc = [jnp.stack([params['pos_c_mlp']['W%d' % k].T,
                     params['neg_c_mlp']['W%d' % k].T]) for k in (1, 2, 3)]
    bc = [jnp.stack([params['pos_c_mlp']['b%d' % k],
                     params['neg_c_mlp']['b%d' % k]])[:, None, :] for k in (1, 2, 3)]
    wih = jnp.stack([params['l_lstm']['W_ih'].T, params['c_lstm']['W_ih'].T])
    whh = jnp.stack([params['l_lstm']['W_hh'].T, params['c_lstm']['W_hh'].T])
    bb = jnp.stack([params['l_lstm']['b_ih'] + params['l_lstm']['b_hh'],
                    params['c_lstm']['b_ih'] + params['c_lstm']['b_hh']])[:, None, :]

    S = jnp.stack([pos_l_emb, neg_l_emb, c_emb])
    C = jnp.zeros_like(S)
    for _ in range(N_ROUND):
        msg_l, msg_c = _msg_call(S, wl, bl, wc, bc)
        l2c = _sc_edge_pass(msg_l, ps_g, pd_s, ns_bg, nd_s)
        c2l = _sc_edge_pass(msg_c, pd_g, ps_s, nd_bg, ns_s)
        S, C = _lstm_call(l2c, c2l, S, C, wih, whh, bb)
    return S.reshape(3 * N_NODE, EMB_D)
